# baseline (speedup 1.0000x reference)
import jax
import jax.numpy as jnp
from jax.experimental import pallas as pl
from jax.experimental.pallas import tpu as pltpu

_E = 8
_T = 32
_H = 4096
_O = 4096
_H_BS = 256


def _moe_body(x_ref, gate_ref, w_ref, out_ref, wts_ref):
    e = pl.program_id(0)
    h = pl.program_id(1)

    @pl.when((e == 0) & (h == 0))
    def _compute_routing():
        logits = jax.lax.dot_general(
            x_ref[:, 0, :], gate_ref[...], (((1,), (1,)), ((), ())),
            preferred_element_type=jnp.float32)
        idx = jax.lax.broadcasted_iota(jnp.int32, (_T, _E), 1)
        m1 = jnp.max(logits, axis=1, keepdims=True)
        i1 = jnp.min(jnp.where(logits == m1, idx, _E), axis=1, keepdims=True)
        masked = jnp.where(idx == i1, -jnp.inf, logits)
        m2 = jnp.max(masked, axis=1, keepdims=True)
        i2 = jnp.min(jnp.where(masked == m2, idx, _E), axis=1, keepdims=True)
        r = jnp.exp(m2 - m1)
        w1 = 1.0 / (1.0 + r)
        w2 = 1.0 - w1
        wts_ref[...] = (jnp.where(idx == i1, w1, 0.0)
                        + jnp.where(idx == i2, w2, 0.0))
        out_ref[...] = jnp.zeros_like(out_ref)

    xs = x_ref[:, 0, pl.ds(h * _H_BS, _H_BS)]
    contrib = jnp.dot(xs, w_ref[0], preferred_element_type=jnp.float32)
    idx = jax.lax.broadcasted_iota(jnp.int32, (_T, _E), 1)
    tw = jnp.sum(jnp.where(idx == e, wts_ref[...], 0.0),
                 axis=1, keepdims=True)
    out_ref[:, 0, :] += contrib * tw


def kernel(x, gate_w, expert_w):
    B_, S_, H = x.shape
    out = pl.pallas_call(
        _moe_body,
        grid=(_E, _H // _H_BS),
        in_specs=[
            pl.BlockSpec((_T, 1, _H), lambda e, h: (0, 0, 0)),
            pl.BlockSpec((_E, _H), lambda e, h: (0, 0)),
            pl.BlockSpec((1, _H_BS, _O), lambda e, h: (e, h, 0)),
        ],
        out_specs=pl.BlockSpec((_T, 1, _O), lambda e, h: (0, 0, 0)),
        out_shape=jax.ShapeDtypeStruct((_T, 1, _O), jnp.float32),
        scratch_shapes=[pltpu.VMEM((_T, _E), jnp.float32)],
        compiler_params=pltpu.CompilerParams(
            dimension_semantics=("arbitrary", "arbitrary")),
    )(x, gate_w.T, expert_w)
    return out


# speedup vs baseline: 1.1802x; 1.1802x over previous
import jax
import jax.numpy as jnp
from jax.experimental import pallas as pl
from jax.experimental.pallas import tpu as pltpu

_E = 8
_T = 32
_H = 4096
_O = 4096
_H_BS = 512


def _moe_body(x_ref, gate_ref, w_ref, out_ref, wts_ref):
    e = pl.program_id(0)
    h = pl.program_id(1)

    @pl.when((e == 0) & (h == 0))
    def _compute_routing():
        logits = jax.lax.dot_general(
            x_ref[:, 0, :], gate_ref[...], (((1,), (1,)), ((), ())),
            preferred_element_type=jnp.float32)
        idx = jax.lax.broadcasted_iota(jnp.int32, (_T, _E), 1)
        m1 = jnp.max(logits, axis=1, keepdims=True)
        i1 = jnp.min(jnp.where(logits == m1, idx, _E), axis=1, keepdims=True)
        masked = jnp.where(idx == i1, -jnp.inf, logits)
        m2 = jnp.max(masked, axis=1, keepdims=True)
        i2 = jnp.min(jnp.where(masked == m2, idx, _E), axis=1, keepdims=True)
        r = jnp.exp(m2 - m1)
        w1 = 1.0 / (1.0 + r)
        w2 = 1.0 - w1
        wts_ref[...] = (jnp.where(idx == i1, w1, 0.0)
                        + jnp.where(idx == i2, w2, 0.0))
        out_ref[...] = jnp.zeros_like(out_ref)

    xs = x_ref[:, 0, pl.ds(h * _H_BS, _H_BS)]
    contrib = jnp.dot(xs, w_ref[0], preferred_element_type=jnp.float32)
    idx = jax.lax.broadcasted_iota(jnp.int32, (_T, _E), 1)
    tw = jnp.sum(jnp.where(idx == e, wts_ref[...], 0.0),
                 axis=1, keepdims=True)
    out_ref[:, 0, :] += contrib * tw


def kernel(x, gate_w, expert_w):
    B_, S_, H = x.shape
    out = pl.pallas_call(
        _moe_body,
        grid=(_E, _H // _H_BS),
        in_specs=[
            pl.BlockSpec((_T, 1, _H), lambda e, h: (0, 0, 0)),
            pl.BlockSpec((_E, _H), lambda e, h: (0, 0)),
            pl.BlockSpec((1, _H_BS, _O), lambda e, h: (e, h, 0)),
        ],
        out_specs=pl.BlockSpec((_T, 1, _O), lambda e, h: (0, 0, 0)),
        out_shape=jax.ShapeDtypeStruct((_T, 1, _O), jnp.float32),
        scratch_shapes=[pltpu.VMEM((_T, _E), jnp.float32)],
        compiler_params=pltpu.CompilerParams(
            dimension_semantics=("arbitrary", "arbitrary")),
    )(x, gate_w.T, expert_w)
    return out
